# baseline (speedup 1.0000x reference)
import functools

import jax
import jax.numpy as jnp
from jax import lax
from jax.experimental import pallas as pl
from jax.experimental.pallas import tpu as pltpu
from jax.experimental.pallas import tpu_sc as plsc

B = 16384
L = 26
EMB = 64
PAD = 128
NUM_DIM = 16
HID = 256
HID2 = 128
OUT = 2

TOKENS = B * L
NW = 32
PER_W = TOKENS // NW
CH = 128
NCH = PER_W // CH

BB = 512
BBL = BB * L
GRID = B // BB


def _gather_body(idx_hbm, table_hbm, out_hbm, idx_v, rows_v, sem0, sem1):
    wid = lax.axis_index("s") * 2 + lax.axis_index("c")
    row0 = wid * NCH
    base = wid * PER_W

    pltpu.sync_copy(idx_hbm.at[pl.ds(row0, NCH)], idx_v)

    sems = (sem0, sem1)

    def start(j, slot):
        pltpu.async_copy(table_hbm.at[idx_v.at[j]], rows_v.at[slot], sems[slot])

    def wait(slot):
        pltpu.make_async_copy(table_hbm.at[idx_v.at[0]], rows_v.at[slot],
                              sems[slot]).wait()

    start(0, 0)
    start(1, 1)

    def pair(i, _):
        j0 = 2 * i
        for s in range(2):
            j = j0 + s
            wait(s)
            pltpu.sync_copy(rows_v.at[s], out_hbm.at[pl.ds(base + j * CH, CH)])

            @pl.when(j + 2 < NCH)
            def _():
                start(j + 2, s)
        return _

    lax.fori_loop(0, NCH // 2, pair, None)


def _sc_gather(idx2d, table_pad):
    mesh = plsc.VectorSubcoreMesh(core_axis_name="c", subcore_axis_name="s")
    k = functools.partial(
        pl.kernel,
        mesh=mesh,
        out_type=jax.ShapeDtypeStruct((TOKENS, PAD), jnp.float32),
        scratch_types=[
            pltpu.VMEM((NCH, CH), jnp.int32),
            pltpu.VMEM((2, CH, PAD), jnp.float32),
            pltpu.SemaphoreType.DMA,
            pltpu.SemaphoreType.DMA,
        ],
    )(_gather_body)
    return k(idx2d, table_pad)


def _mlp_body(emb_ref, num_ref, w1a_ref, w1b_ref, b1_ref, w2_ref, b2_ref,
              w3_ref, b3_ref, out_ref):
    f32 = jnp.float32
    emb = emb_ref[...][:, :EMB]
    h = jnp.dot(emb, w1a_ref[...], preferred_element_type=f32)
    h += jnp.dot(num_ref[...], w1b_ref[...], preferred_element_type=f32)
    h += b1_ref[...]
    h = jnp.maximum(h, 0.0)
    h = jnp.dot(h, w2_ref[...], preferred_element_type=f32) + b2_ref[...]
    h = jnp.maximum(h, 0.0)
    z = jnp.dot(h, w3_ref[...], preferred_element_type=f32) + b3_ref[...]
    z3 = z.reshape(BB, L, OUT)
    m = jnp.max(z3, axis=1, keepdims=True)
    e = jnp.exp(z3 - m)
    s = jnp.sum(e, axis=1, keepdims=True)
    out_ref[...] = e / s


def _tc_mlp(gathered, num_tok, W1a, W1b, b1, W2, b2, W3, b3):
    return pl.pallas_call(
        _mlp_body,
        grid=(GRID,),
        in_specs=[
            pl.BlockSpec((BBL, PAD), lambda i: (i, 0)),
            pl.BlockSpec((BBL, NUM_DIM), lambda i: (i, 0)),
            pl.BlockSpec((EMB, HID), lambda i: (0, 0)),
            pl.BlockSpec((NUM_DIM, HID), lambda i: (0, 0)),
            pl.BlockSpec((1, HID), lambda i: (0, 0)),
            pl.BlockSpec((HID, HID2), lambda i: (0, 0)),
            pl.BlockSpec((1, HID2), lambda i: (0, 0)),
            pl.BlockSpec((HID2, OUT), lambda i: (0, 0)),
            pl.BlockSpec((1, OUT), lambda i: (0, 0)),
        ],
        out_specs=pl.BlockSpec((BB, L, OUT), lambda i: (i, 0, 0)),
        out_shape=jax.ShapeDtypeStruct((B, L, OUT), jnp.float32),
        compiler_params=pltpu.CompilerParams(
            dimension_semantics=("arbitrary",),
        ),
    )(gathered, num_tok, W1a, W1b, b1, W2, b2, W3, b3)


def kernel(categorical_data, numerical_data, emb_table, W1, b1, W2, b2, W3, b3):
    idx = categorical_data.astype(jnp.int32).reshape(TOKENS)
    idx2d = idx.reshape(NW * NCH, CH)
    table_pad = jnp.pad(emb_table, ((0, 0), (0, PAD - EMB)))
    gathered = _sc_gather(idx2d, table_pad)

    num_tok = jnp.repeat(numerical_data, L, axis=0)
    W1a = W1[:EMB]
    W1b = W1[EMB:]
    out = _tc_mlp(gathered, num_tok, W1a, W1b, b1.reshape(1, HID),
                  W2, b2.reshape(1, HID2), W3, b3.reshape(1, OUT))
    return out


# speedup vs baseline: 1.2084x; 1.2084x over previous
import functools

import jax
import jax.numpy as jnp
from jax import lax
from jax.experimental import pallas as pl
from jax.experimental.pallas import tpu as pltpu
from jax.experimental.pallas import tpu_sc as plsc

B = 16384
L = 26
EMB = 64
PAD = 128
NUM_DIM = 16
HID = 256
HID2 = 128
OUT = 2
LO = L * OUT

TOKENS = B * L
NW = 32
PER_W = TOKENS // NW
CH = 128
NCH = PER_W // CH

BB = 512
BBL = BB * L
GRID = B // BB

SB = 2048
SGRID = B // SB


def _gather_body(idx_hbm, table_hbm, out_hbm, idx_v, rows_v, sem0, sem1):
    wid = lax.axis_index("s") * 2 + lax.axis_index("c")
    row0 = wid * NCH
    base = wid * PER_W

    pltpu.sync_copy(idx_hbm.at[pl.ds(row0, NCH)], idx_v)

    sems = (sem0, sem1)

    def start(j, slot):
        pltpu.async_copy(table_hbm.at[idx_v.at[j]], rows_v.at[slot], sems[slot])

    def wait(slot):
        pltpu.make_async_copy(table_hbm.at[idx_v.at[0]], rows_v.at[slot],
                              sems[slot]).wait()

    start(0, 0)
    start(1, 1)

    def pair(i, _):
        j0 = 2 * i
        for s in range(2):
            j = j0 + s
            wait(s)
            pltpu.sync_copy(rows_v.at[s], out_hbm.at[pl.ds(base + j * CH, CH)])

            @pl.when(j + 2 < NCH)
            def _():
                start(j + 2, s)
        return _

    lax.fori_loop(0, NCH // 2, pair, None)


def _sc_gather(idx2d, table_pad):
    mesh = plsc.VectorSubcoreMesh(core_axis_name="c", subcore_axis_name="s")
    k = functools.partial(
        pl.kernel,
        mesh=mesh,
        out_type=jax.ShapeDtypeStruct((TOKENS, PAD), jnp.float32),
        scratch_types=[
            pltpu.VMEM((NCH, CH), jnp.int32),
            pltpu.VMEM((2, CH, PAD), jnp.float32),
            pltpu.SemaphoreType.DMA,
            pltpu.SemaphoreType.DMA,
        ],
    )(_gather_body)
    return k(idx2d, table_pad)


def _mlp_body(emb_ref, num_ref, w1a_ref, w1b_ref, b1_ref, w2_ref, b2_ref,
              w3_ref, b3_ref, z_ref):
    f32 = jnp.float32
    h = jnp.dot(emb_ref[...][:, :EMB], w1a_ref[...], preferred_element_type=f32)
    h += jnp.dot(num_ref[...], w1b_ref[...], preferred_element_type=f32)
    h += b1_ref[...]
    h = jnp.maximum(h, 0.0)
    h = jnp.dot(h, w2_ref[...], preferred_element_type=f32) + b2_ref[...]
    h = jnp.maximum(h, 0.0)
    z_ref[...] = jnp.dot(h, w3_ref[...], preferred_element_type=f32) + b3_ref[...]


def _tc_mlp(gathered, num_tok, W1a, W1b, b1, W2, b2, W3, b3):
    return pl.pallas_call(
        _mlp_body,
        grid=(GRID,),
        in_specs=[
            pl.BlockSpec((BBL, PAD), lambda i: (i, 0)),
            pl.BlockSpec((BBL, NUM_DIM), lambda i: (i, 0)),
            pl.BlockSpec((EMB, HID), lambda i: (0, 0)),
            pl.BlockSpec((NUM_DIM, HID), lambda i: (0, 0)),
            pl.BlockSpec((1, HID), lambda i: (0, 0)),
            pl.BlockSpec((HID, HID2), lambda i: (0, 0)),
            pl.BlockSpec((1, HID2), lambda i: (0, 0)),
            pl.BlockSpec((HID2, OUT), lambda i: (0, 0)),
            pl.BlockSpec((1, OUT), lambda i: (0, 0)),
        ],
        out_specs=pl.BlockSpec((BBL, OUT), lambda i: (i, 0)),
        out_shape=jax.ShapeDtypeStruct((TOKENS, OUT), jnp.float32),
        compiler_params=pltpu.CompilerParams(
            dimension_semantics=("arbitrary",),
        ),
    )(gathered, num_tok, W1a, W1b, b1, W2, b2, W3, b3)


def _softmax_body(z_ref, o_ref):
    z = z_ref[...]
    m = jnp.max(z, axis=1, keepdims=True)
    e = jnp.exp(z - m)
    ko = lax.broadcasted_iota(jnp.int32, (LO, OUT), 0)
    oo = lax.broadcasted_iota(jnp.int32, (LO, OUT), 1)
    sel = jnp.where((ko % OUT) == oo, 1.0, 0.0)
    denom = jnp.dot(e, sel, preferred_element_type=jnp.float32)
    dexp = jnp.dot(denom, sel.T, preferred_element_type=jnp.float32)
    o_ref[...] = e / dexp


def _tc_softmax(z2d):
    return pl.pallas_call(
        _softmax_body,
        grid=(SGRID,),
        in_specs=[pl.BlockSpec((SB, LO), lambda i: (i, 0))],
        out_specs=pl.BlockSpec((SB, LO), lambda i: (i, 0)),
        out_shape=jax.ShapeDtypeStruct((B, LO), jnp.float32),
        compiler_params=pltpu.CompilerParams(
            dimension_semantics=("arbitrary",),
        ),
    )(z2d)


def kernel(categorical_data, numerical_data, emb_table, W1, b1, W2, b2, W3, b3):
    idx = categorical_data.astype(jnp.int32).reshape(TOKENS)
    idx2d = idx.reshape(NW * NCH, CH)
    table_pad = jnp.pad(emb_table, ((0, 0), (0, PAD - EMB)))
    gathered = _sc_gather(idx2d, table_pad)

    num_tok = jnp.repeat(numerical_data, L, axis=0)
    W1a = W1[:EMB]
    W1b = W1[EMB:]
    z = _tc_mlp(gathered, num_tok, W1a, W1b, b1.reshape(1, HID),
                W2, b2.reshape(1, HID2), W3, b3.reshape(1, OUT))
    sm = _tc_softmax(z.reshape(B, LO))
    return sm.reshape(B, L, OUT)
